# Initial kernel scaffold; baseline (speedup 1.0000x reference)
import jax
import jax.numpy as jnp
from jax.experimental import pallas as pl


def kernel(users, pos_items, neg_items, user_emb, item_emb, W_gc_1, b_gc_1, W_gc, b_gc, adj_rows, adj_cols, adj_vals):
    raise NotImplementedError("write your pallas kernel here")


# speedup vs baseline: 4.5879x; 4.5879x over previous
import functools

import jax
import jax.numpy as jnp
from jax import lax
from jax.experimental import pallas as pl
from jax.experimental.pallas import tpu as pltpu
from jax.experimental.pallas import tpu_sc as plsc

NU = 6000
NI = 4000
NN = 10000
NP = 10240
D = 256
DH = 128
E = 160000
GR = 8
B = 1024
RB = 640
NRB = NP // RB


_TILES = 16
_EPT = E // _TILES
_CHUNK = 80
_NCHUNK = _EPT // _CHUNK
_RPT = NP // _TILES

@functools.lru_cache(maxsize=None)
def _sc_mesh():
    return plsc.VectorSubcoreMesh(core_axis_name="c", subcore_axis_name="s")


def _spmm_body(rows_h, cols_h, vals_h, x0_h, x1_h, zeros_h, out0_h, out1_h,
               idxr_v, idxc_v, vals_v, gath_v, acc_sh, sem):
    c = lax.axis_index("c")
    s = lax.axis_index("s")
    rbase = s * _RPT
    pltpu.sync_copy(zeros_h.at[pl.ds(rbase, _RPT)], acc_sh.at[pl.ds(rbase, _RPT)])
    plsc.subcore_barrier()

    ebase = s * _EPT

    def chunk_body(k, carry):
        off = ebase + k * _CHUNK
        pltpu.sync_copy(rows_h.at[pl.ds(off, _CHUNK)], idxr_v)
        pltpu.sync_copy(cols_h.at[pl.ds(off, _CHUNK)], idxc_v)
        pltpu.sync_copy(vals_h.at[pl.ds(off * 16, _CHUNK * 16)], vals_v)

        @pl.when(c == 0)
        def _():
            pltpu.async_copy(x0_h.at[idxc_v], gath_v, sem).wait()

        @pl.when(c == 1)
        def _():
            pltpu.async_copy(x1_h.at[idxc_v], gath_v, sem).wait()

        def row_body(i, carry2):
            v = vals_v[pl.ds(i * 16, 16)]
            for d in range(DH // 16):
                sl = pl.ds(d * 16, 16)
                gath_v[i, sl] = gath_v[i, sl] * v
            return carry2

        lax.fori_loop(0, _CHUNK, row_body, 0)
        pltpu.sync_copy(gath_v, acc_sh.at[idxr_v], add=True)
        return carry

    lax.fori_loop(0, _NCHUNK, chunk_body, 0)
    plsc.subcore_barrier()

    @pl.when(c == 0)
    def _():
        pltpu.sync_copy(acc_sh.at[pl.ds(rbase, _RPT)], out0_h.at[pl.ds(rbase, _RPT)])

    @pl.when(c == 1)
    def _():
        pltpu.sync_copy(acc_sh.at[pl.ds(rbase, _RPT)], out1_h.at[pl.ds(rbase, _RPT)])


@functools.lru_cache(maxsize=None)
def _spmm_kernel():
    return pl.kernel(
        _spmm_body,
        out_type=[
            jax.ShapeDtypeStruct((NP, DH), jnp.float32),
            jax.ShapeDtypeStruct((NP, DH), jnp.float32),
        ],
        mesh=_sc_mesh(),
        scratch_types=[
            pltpu.VMEM((_CHUNK,), jnp.int32),
            pltpu.VMEM((_CHUNK,), jnp.int32),
            pltpu.VMEM((_CHUNK * 16,), jnp.float32),
            pltpu.VMEM((_CHUNK, DH), jnp.float32),
            pltpu.VMEM_SHARED((NP, DH), jnp.float32),
            pltpu.SemaphoreType.DMA,
        ],
    )


def _spmm(rows, cols, valsb, x0, x1, zeros):
    return _spmm_kernel()(rows, cols, valsb, x0, x1, zeros)


def _route_body(e0, e1, s0, s1, w1a, w1b, b1, w2p, b2p, g_ref):
    xa = e0[...] + s0[...]
    xb = e1[...] + s1[...]
    h = (jnp.dot(xa, w1a[...], preferred_element_type=jnp.float32)
         + jnp.dot(xb, w1b[...], preferred_element_type=jnp.float32)
         + b1[0:1, :])
    h = jnp.where(h >= 0, h, 0.01 * h)
    gs = jnp.dot(h, w2p[...], preferred_element_type=jnp.float32) + b2p[0:1, :]
    m = jnp.max(gs, axis=1, keepdims=True)
    g = (gs == m).astype(jnp.float32)
    row = pl.program_id(0) * RB + lax.broadcasted_iota(jnp.int32, (RB, 128), 0)
    g_ref[...] = jnp.where(row < NU, g, 1.0)


def _route(e0, e1, s0, s1, w1a, w1b, b1, w2p, b2p):
    blk = pl.BlockSpec((RB, DH), lambda i: (i, 0))
    full = lambda a: pl.BlockSpec(a.shape, lambda i: tuple(0 for _ in a.shape))
    return pl.pallas_call(
        _route_body,
        grid=(NRB,),
        in_specs=[blk, blk, blk, blk, full(w1a), full(w1b), full(b1),
                  full(w2p), full(b2p)],
        out_specs=pl.BlockSpec((RB, 128), lambda i: (i, 0)),
        out_shape=jax.ShapeDtypeStruct((NP, 128), jnp.float32),
    )(e0, e1, s0, s1, w1a, w1b, b1, w2p, b2p)


def _mask_body(g_ref, e0, e1, x0_ref, x1_ref):
    g = pl.program_id(0)
    onehot = (lax.broadcasted_iota(jnp.int32, (RB, 128), 1) == g).astype(jnp.float32)
    col = jnp.sum(g_ref[...] * onehot, axis=1, keepdims=True)
    x0_ref[0] = col * e0[...]
    x1_ref[0] = col * e1[...]


def _mask(G, e0, e1):
    blk = pl.BlockSpec((RB, DH), lambda g, r: (r, 0))
    gblk = pl.BlockSpec((RB, 128), lambda g, r: (r, 0))
    oblk = pl.BlockSpec((1, RB, DH), lambda g, r: (g, r, 0))
    return pl.pallas_call(
        _mask_body,
        grid=(GR, NRB),
        in_specs=[gblk, blk, blk],
        out_specs=[oblk, oblk],
        out_shape=[jax.ShapeDtypeStruct((GR, NP, DH), jnp.float32),
                   jax.ShapeDtypeStruct((GR, NP, DH), jnp.float32)],
    )(G, e0, e1)


def _elem1_body(e0, e1, y0, y1, g_ref, s1_0, s1_1, x2_0, x2_1):
    g = pl.program_id(1)
    onehot = (lax.broadcasted_iota(jnp.int32, (RB, 128), 1) == g).astype(jnp.float32)
    col = jnp.sum(g_ref[...] * onehot, axis=1, keepdims=True)
    ea = e0[...]
    eb = e1[...]
    mya = col * y0[0]
    myb = col * y1[0]
    fa = ea + mya
    fb = eb + myb
    dot = jnp.sum(fa * ea, axis=1, keepdims=True) + jnp.sum(fb * eb, axis=1, keepdims=True)
    na = jnp.sqrt(jnp.sum(fa * fa, axis=1, keepdims=True) + jnp.sum(fb * fb, axis=1, keepdims=True))
    nb = jnp.sqrt(jnp.sum(ea * ea, axis=1, keepdims=True) + jnp.sum(eb * eb, axis=1, keepdims=True))
    w = dot / (jnp.maximum(na, 1e-8) * jnp.maximum(nb, 1e-8))
    x2_0[0] = w * (col * ea + mya)
    x2_1[0] = w * (col * eb + myb)

    @pl.when(g == 0)
    def _():
        s1_0[...] = mya
        s1_1[...] = myb

    @pl.when(g > 0)
    def _():
        s1_0[...] += mya
        s1_1[...] += myb


def _elem1(e0, e1, y0, y1, G):
    blk = pl.BlockSpec((RB, DH), lambda r, g: (r, 0))
    gblk = pl.BlockSpec((RB, 128), lambda r, g: (r, 0))
    ybk = pl.BlockSpec((1, RB, DH), lambda r, g: (g, r, 0))
    return pl.pallas_call(
        _elem1_body,
        grid=(NRB, GR),
        in_specs=[blk, blk, ybk, ybk, gblk],
        out_specs=[blk, blk, ybk, ybk],
        out_shape=[jax.ShapeDtypeStruct((NP, DH), jnp.float32),
                   jax.ShapeDtypeStruct((NP, DH), jnp.float32),
                   jax.ShapeDtypeStruct((GR, NP, DH), jnp.float32),
                   jax.ShapeDtypeStruct((GR, NP, DH), jnp.float32)],
    )(e0, e1, y0, y1, G)


def _elem2_body(s0, s1, g_ref, o0, o1):
    g = pl.program_id(1)
    onehot = (lax.broadcasted_iota(jnp.int32, (RB, 128), 1) == g).astype(jnp.float32)
    col = jnp.sum(g_ref[...] * onehot, axis=1, keepdims=True)
    va = col * s0[0]
    vb = col * s1[0]

    @pl.when(g == 0)
    def _():
        o0[...] = va
        o1[...] = vb

    @pl.when(g > 0)
    def _():
        o0[...] += va
        o1[...] += vb


def _elem2(s2_0, s2_1, G):
    blk = pl.BlockSpec((RB, DH), lambda r, g: (r, 0))
    gblk = pl.BlockSpec((RB, 128), lambda r, g: (r, 0))
    sbk = pl.BlockSpec((1, RB, DH), lambda r, g: (g, r, 0))
    return pl.pallas_call(
        _elem2_body,
        grid=(NRB, GR),
        in_specs=[sbk, sbk, gblk],
        out_specs=[blk, blk],
        out_shape=[jax.ShapeDtypeStruct((NP, DH), jnp.float32),
                   jax.ShapeDtypeStruct((NP, DH), jnp.float32)],
    )(s2_0, s2_1, G)


def _final_body(e0, e1, s0, s1, c10, c11, c20, c21, f0, f1):
    f0[...] = e0[...] + s0[...] + c10[...] + c20[...]
    f1[...] = e1[...] + s1[...] + c11[...] + c21[...]


def _final(e0, e1, s0, s1, c10, c11, c20, c21):
    blk = pl.BlockSpec((RB, DH), lambda r: (r, 0))
    return pl.pallas_call(
        _final_body,
        grid=(NRB,),
        in_specs=[blk] * 8,
        out_specs=[blk, blk],
        out_shape=[jax.ShapeDtypeStruct((NP, DH), jnp.float32),
                   jax.ShapeDtypeStruct((NP, DH), jnp.float32)],
    )(e0, e1, s0, s1, c10, c11, c20, c21)


_GB = 3 * B
_GPW = _GB // 32


def _gather_body(f0_h, f1_h, idx_h, out_h, idx_v, r0_v, r1_v, sem):
    c = lax.axis_index("c")
    s = lax.axis_index("s")
    base = (s * 2 + c) * _GPW
    pltpu.sync_copy(idx_h.at[pl.ds(base, _GPW)], idx_v)
    pltpu.async_copy(f0_h.at[idx_v], r0_v, sem).wait()
    pltpu.async_copy(f1_h.at[idx_v], r1_v, sem).wait()
    pltpu.sync_copy(r0_v, out_h.at[0, pl.ds(base, _GPW)])
    pltpu.sync_copy(r1_v, out_h.at[1, pl.ds(base, _GPW)])


@functools.lru_cache(maxsize=None)
def _gather_kernel():
    return pl.kernel(
        _gather_body,
        out_type=jax.ShapeDtypeStruct((2, _GB, DH), jnp.float32),
        mesh=_sc_mesh(),
        scratch_types=[
            pltpu.VMEM((_GPW,), jnp.int32),
            pltpu.VMEM((_GPW, DH), jnp.float32),
            pltpu.VMEM((_GPW, DH), jnp.float32),
            pltpu.SemaphoreType.DMA,
        ],
    )


def kernel(users, pos_items, neg_items, user_emb, item_emb,
           W_gc_1, b_gc_1, W_gc, b_gc, adj_rows, adj_cols, adj_vals):
    f32 = jnp.float32
    rows = adj_rows.astype(jnp.int32)
    cols = adj_cols.astype(jnp.int32)
    vals = adj_vals.astype(f32)
    valsb = jnp.reshape(jnp.broadcast_to(vals[:, None], (E, 16)), (E * 16,))

    pad = jnp.zeros((NP - NN, DH), f32)
    e0 = jnp.concatenate([user_emb[:, :DH], item_emb[:, :DH], pad], axis=0)
    e1 = jnp.concatenate([user_emb[:, DH:], item_emb[:, DH:], pad], axis=0)
    zeros = jnp.zeros((NP, DH), f32)

    w1a = W_gc_1[:DH, :]
    w1b = W_gc_1[DH:, :]
    b1 = jnp.broadcast_to(b_gc_1, (8, D))
    w2p = jnp.concatenate([W_gc, jnp.zeros((D, 128 - GR), f32)], axis=1)
    b2p = jnp.broadcast_to(
        jnp.concatenate([b_gc, jnp.full((1, 128 - GR), -1e30, f32)], axis=1),
        (8, 128))

    s0, s1 = _spmm(rows, cols, valsb, e0, e1, zeros)
    G = _route(e0, e1, s0, s1, w1a, w1b, b1, w2p, b2p)

    x8_0, x8_1 = _mask(G, e0, e1)
    ys = [_spmm(rows, cols, valsb, x8_0[g], x8_1[g], zeros) for g in range(GR)]
    y0 = jnp.stack([y[0] for y in ys])
    y1 = jnp.stack([y[1] for y in ys])

    sum1_0, sum1_1, x2_0, x2_1 = _elem1(e0, e1, y0, y1, G)
    c10, c11 = _spmm(rows, cols, valsb, sum1_0, sum1_1, zeros)

    s2s = [_spmm(rows, cols, valsb, x2_0[g], x2_1[g], zeros) for g in range(GR)]
    s2_0 = jnp.stack([s[0] for s in s2s])
    s2_1 = jnp.stack([s[1] for s in s2s])
    sum2_0, sum2_1 = _elem2(s2_0, s2_1, G)
    c20, c21 = _spmm(rows, cols, valsb, sum2_0, sum2_1, zeros)

    f0, f1 = _final(e0, e1, s0, s1, c10, c11, c20, c21)

    idx = jnp.concatenate([users.astype(jnp.int32),
                           NU + pos_items.astype(jnp.int32),
                           NU + neg_items.astype(jnp.int32)])
    go = _gather_kernel()(f0, f1, idx)
    o = jnp.concatenate([go[0], go[1]], axis=1)
    return (o[:B], o[B:2 * B], o[2 * B:])


# speedup vs baseline: 9.1408x; 1.9924x over previous
import functools

import jax
import jax.numpy as jnp
from jax import lax
from jax.experimental import pallas as pl
from jax.experimental.pallas import tpu as pltpu
from jax.experimental.pallas import tpu_sc as plsc

NU = 6000
NI = 4000
NN = 10000
NP = 10240
D = 256
DH = 128
E = 160000
GR = 8
B = 1024
RB = 640
NRB = NP // RB


_TILES = 16
_EPT = E // _TILES
_CHUNK = 80
_NCHUNK = _EPT // _CHUNK
_RPT = NP // _TILES

@functools.lru_cache(maxsize=None)
def _sc_mesh():
    return plsc.VectorSubcoreMesh(core_axis_name="c", subcore_axis_name="s")


def _spmm_body(rows_h, cols_h, vals_h, x0_h, x1_h, zeros_h, out0_h, out1_h,
               idxr0, idxr1, idxc0, idxc1, vals0, vals1, gath0, gath1,
               acc_sh, semg0, semg1, sema0, sema1):
    c = lax.axis_index("c")
    s = lax.axis_index("s")
    rbase = s * _RPT
    ebase = s * _EPT
    idxr = (idxr0, idxr1)
    idxc = (idxc0, idxc1)
    vals = (vals0, vals1)
    gath = (gath0, gath1)
    semg = (semg0, semg1)
    sema = (sema0, sema1)

    def prefetch(b, k):
        off = ebase + k * _CHUNK
        pltpu.sync_copy(cols_h.at[pl.ds(off, _CHUNK)], idxc[b])

        @pl.when(c == 0)
        def _():
            pltpu.async_copy(x0_h.at[idxc[b]], gath[b], semg[b])

        @pl.when(c == 1)
        def _():
            pltpu.async_copy(x1_h.at[idxc[b]], gath[b], semg[b])

        pltpu.async_copy(rows_h.at[pl.ds(off, _CHUNK)], idxr[b], sema[b])
        pltpu.async_copy(vals_h.at[pl.ds(off * 16, _CHUNK * 16)], vals[b], sema[b])

    def wait_bufs(b):
        pltpu.make_async_copy(x0_h.at[idxc[b]], gath[b], semg[b]).wait()
        pltpu.make_async_copy(rows_h.at[pl.ds(0, _CHUNK)], idxr[b], sema[b]).wait()
        pltpu.make_async_copy(vals_h.at[pl.ds(0, _CHUNK * 16)], vals[b], sema[b]).wait()

    def process(b):
        def row_body(i, carry2):
            v = vals[b][pl.ds(i * 16, 16)]
            for d in range(DH // 16):
                sl = pl.ds(d * 16, 16)
                gath[b][i, sl] = gath[b][i, sl] * v
            return carry2

        lax.fori_loop(0, _CHUNK, row_body, 0)
        pltpu.sync_copy(gath[b], acc_sh.at[idxr[b]], add=True)

    prefetch(0, 0)
    pltpu.sync_copy(zeros_h.at[pl.ds(rbase, _RPT)], acc_sh.at[pl.ds(rbase, _RPT)])
    plsc.subcore_barrier()

    def pair_body(p, carry):
        wait_bufs(0)
        prefetch(1, 2 * p + 1)
        process(0)
        wait_bufs(1)
        prefetch(0, 2 * p + 2)
        process(1)
        return carry

    lax.fori_loop(0, (_NCHUNK - 1) // 2, pair_body, 0)
    wait_bufs(0)
    process(0)
    plsc.subcore_barrier()

    @pl.when(c == 0)
    def _():
        pltpu.sync_copy(acc_sh.at[pl.ds(rbase, _RPT)], out0_h.at[pl.ds(rbase, _RPT)])

    @pl.when(c == 1)
    def _():
        pltpu.sync_copy(acc_sh.at[pl.ds(rbase, _RPT)], out1_h.at[pl.ds(rbase, _RPT)])


@functools.lru_cache(maxsize=None)
def _spmm_kernel():
    return pl.kernel(
        _spmm_body,
        out_type=[
            jax.ShapeDtypeStruct((NP, DH), jnp.float32),
            jax.ShapeDtypeStruct((NP, DH), jnp.float32),
        ],
        mesh=_sc_mesh(),
        scratch_types=[
            pltpu.VMEM((_CHUNK,), jnp.int32),
            pltpu.VMEM((_CHUNK,), jnp.int32),
            pltpu.VMEM((_CHUNK,), jnp.int32),
            pltpu.VMEM((_CHUNK,), jnp.int32),
            pltpu.VMEM((_CHUNK * 16,), jnp.float32),
            pltpu.VMEM((_CHUNK * 16,), jnp.float32),
            pltpu.VMEM((_CHUNK, DH), jnp.float32),
            pltpu.VMEM((_CHUNK, DH), jnp.float32),
            pltpu.VMEM_SHARED((NP, DH), jnp.float32),
            pltpu.SemaphoreType.DMA,
            pltpu.SemaphoreType.DMA,
            pltpu.SemaphoreType.DMA,
            pltpu.SemaphoreType.DMA,
        ],
    )


def _spmm(rows, cols, valsb, x0, x1, zeros):
    return _spmm_kernel()(rows, cols, valsb, x0, x1, zeros)


def _route_body(e0, e1, s0, s1, w1a, w1b, b1, w2p, b2p, g_ref):
    xa = e0[...] + s0[...]
    xb = e1[...] + s1[...]
    h = (jnp.dot(xa, w1a[...], preferred_element_type=jnp.float32)
         + jnp.dot(xb, w1b[...], preferred_element_type=jnp.float32)
         + b1[0:1, :])
    h = jnp.where(h >= 0, h, 0.01 * h)
    gs = jnp.dot(h, w2p[...], preferred_element_type=jnp.float32) + b2p[0:1, :]
    m = jnp.max(gs, axis=1, keepdims=True)
    g = (gs == m).astype(jnp.float32)
    row = pl.program_id(0) * RB + lax.broadcasted_iota(jnp.int32, (RB, 128), 0)
    g_ref[...] = jnp.where(row < NU, g, 1.0)


def _route(e0, e1, s0, s1, w1a, w1b, b1, w2p, b2p):
    blk = pl.BlockSpec((RB, DH), lambda i: (i, 0))
    full = lambda a: pl.BlockSpec(a.shape, lambda i: tuple(0 for _ in a.shape))
    return pl.pallas_call(
        _route_body,
        grid=(NRB,),
        in_specs=[blk, blk, blk, blk, full(w1a), full(w1b), full(b1),
                  full(w2p), full(b2p)],
        out_specs=pl.BlockSpec((RB, 128), lambda i: (i, 0)),
        out_shape=jax.ShapeDtypeStruct((NP, 128), jnp.float32),
    )(e0, e1, s0, s1, w1a, w1b, b1, w2p, b2p)


def _mask_body(g_ref, e0, e1, x0_ref, x1_ref):
    g = pl.program_id(0)
    onehot = (lax.broadcasted_iota(jnp.int32, (RB, 128), 1) == g).astype(jnp.float32)
    col = jnp.sum(g_ref[...] * onehot, axis=1, keepdims=True)
    x0_ref[0] = col * e0[...]
    x1_ref[0] = col * e1[...]


def _mask(G, e0, e1):
    blk = pl.BlockSpec((RB, DH), lambda g, r: (r, 0))
    gblk = pl.BlockSpec((RB, 128), lambda g, r: (r, 0))
    oblk = pl.BlockSpec((1, RB, DH), lambda g, r: (g, r, 0))
    return pl.pallas_call(
        _mask_body,
        grid=(GR, NRB),
        in_specs=[gblk, blk, blk],
        out_specs=[oblk, oblk],
        out_shape=[jax.ShapeDtypeStruct((GR, NP, DH), jnp.float32),
                   jax.ShapeDtypeStruct((GR, NP, DH), jnp.float32)],
    )(G, e0, e1)


def _elem1_body(e0, e1, y0, y1, g_ref, s1_0, s1_1, x2_0, x2_1):
    g = pl.program_id(1)
    onehot = (lax.broadcasted_iota(jnp.int32, (RB, 128), 1) == g).astype(jnp.float32)
    col = jnp.sum(g_ref[...] * onehot, axis=1, keepdims=True)
    ea = e0[...]
    eb = e1[...]
    mya = col * y0[0]
    myb = col * y1[0]
    fa = ea + mya
    fb = eb + myb
    dot = jnp.sum(fa * ea, axis=1, keepdims=True) + jnp.sum(fb * eb, axis=1, keepdims=True)
    na = jnp.sqrt(jnp.sum(fa * fa, axis=1, keepdims=True) + jnp.sum(fb * fb, axis=1, keepdims=True))
    nb = jnp.sqrt(jnp.sum(ea * ea, axis=1, keepdims=True) + jnp.sum(eb * eb, axis=1, keepdims=True))
    w = dot / (jnp.maximum(na, 1e-8) * jnp.maximum(nb, 1e-8))
    x2_0[0] = w * (col * ea + mya)
    x2_1[0] = w * (col * eb + myb)

    @pl.when(g == 0)
    def _():
        s1_0[...] = mya
        s1_1[...] = myb

    @pl.when(g > 0)
    def _():
        s1_0[...] += mya
        s1_1[...] += myb


def _elem1(e0, e1, y0, y1, G):
    blk = pl.BlockSpec((RB, DH), lambda r, g: (r, 0))
    gblk = pl.BlockSpec((RB, 128), lambda r, g: (r, 0))
    ybk = pl.BlockSpec((1, RB, DH), lambda r, g: (g, r, 0))
    return pl.pallas_call(
        _elem1_body,
        grid=(NRB, GR),
        in_specs=[blk, blk, ybk, ybk, gblk],
        out_specs=[blk, blk, ybk, ybk],
        out_shape=[jax.ShapeDtypeStruct((NP, DH), jnp.float32),
                   jax.ShapeDtypeStruct((NP, DH), jnp.float32),
                   jax.ShapeDtypeStruct((GR, NP, DH), jnp.float32),
                   jax.ShapeDtypeStruct((GR, NP, DH), jnp.float32)],
    )(e0, e1, y0, y1, G)


def _elem2_body(s0, s1, g_ref, o0, o1):
    g = pl.program_id(1)
    onehot = (lax.broadcasted_iota(jnp.int32, (RB, 128), 1) == g).astype(jnp.float32)
    col = jnp.sum(g_ref[...] * onehot, axis=1, keepdims=True)
    va = col * s0[0]
    vb = col * s1[0]

    @pl.when(g == 0)
    def _():
        o0[...] = va
        o1[...] = vb

    @pl.when(g > 0)
    def _():
        o0[...] += va
        o1[...] += vb


def _elem2(s2_0, s2_1, G):
    blk = pl.BlockSpec((RB, DH), lambda r, g: (r, 0))
    gblk = pl.BlockSpec((RB, 128), lambda r, g: (r, 0))
    sbk = pl.BlockSpec((1, RB, DH), lambda r, g: (g, r, 0))
    return pl.pallas_call(
        _elem2_body,
        grid=(NRB, GR),
        in_specs=[sbk, sbk, gblk],
        out_specs=[blk, blk],
        out_shape=[jax.ShapeDtypeStruct((NP, DH), jnp.float32),
                   jax.ShapeDtypeStruct((NP, DH), jnp.float32)],
    )(s2_0, s2_1, G)


def _final_body(e0, e1, s0, s1, c10, c11, c20, c21, f0, f1):
    f0[...] = e0[...] + s0[...] + c10[...] + c20[...]
    f1[...] = e1[...] + s1[...] + c11[...] + c21[...]


def _final(e0, e1, s0, s1, c10, c11, c20, c21):
    blk = pl.BlockSpec((RB, DH), lambda r: (r, 0))
    return pl.pallas_call(
        _final_body,
        grid=(NRB,),
        in_specs=[blk] * 8,
        out_specs=[blk, blk],
        out_shape=[jax.ShapeDtypeStruct((NP, DH), jnp.float32),
                   jax.ShapeDtypeStruct((NP, DH), jnp.float32)],
    )(e0, e1, s0, s1, c10, c11, c20, c21)


_GB = 3 * B
_GPW = _GB // 32


def _gather_body(f0_h, f1_h, idx_h, out_h, idx_v, r0_v, r1_v, sem):
    c = lax.axis_index("c")
    s = lax.axis_index("s")
    base = (s * 2 + c) * _GPW
    pltpu.sync_copy(idx_h.at[pl.ds(base, _GPW)], idx_v)
    pltpu.async_copy(f0_h.at[idx_v], r0_v, sem).wait()
    pltpu.async_copy(f1_h.at[idx_v], r1_v, sem).wait()
    pltpu.sync_copy(r0_v, out_h.at[0, pl.ds(base, _GPW)])
    pltpu.sync_copy(r1_v, out_h.at[1, pl.ds(base, _GPW)])


@functools.lru_cache(maxsize=None)
def _gather_kernel():
    return pl.kernel(
        _gather_body,
        out_type=jax.ShapeDtypeStruct((2, _GB, DH), jnp.float32),
        mesh=_sc_mesh(),
        scratch_types=[
            pltpu.VMEM((_GPW,), jnp.int32),
            pltpu.VMEM((_GPW, DH), jnp.float32),
            pltpu.VMEM((_GPW, DH), jnp.float32),
            pltpu.SemaphoreType.DMA,
        ],
    )


def kernel(users, pos_items, neg_items, user_emb, item_emb,
           W_gc_1, b_gc_1, W_gc, b_gc, adj_rows, adj_cols, adj_vals):
    f32 = jnp.float32
    rows = adj_rows.astype(jnp.int32)
    cols = adj_cols.astype(jnp.int32)
    vals = adj_vals.astype(f32)
    valsb = jnp.reshape(jnp.broadcast_to(vals[:, None], (E, 16)), (E * 16,))

    pad = jnp.zeros((NP - NN, DH), f32)
    e0 = jnp.concatenate([user_emb[:, :DH], item_emb[:, :DH], pad], axis=0)
    e1 = jnp.concatenate([user_emb[:, DH:], item_emb[:, DH:], pad], axis=0)
    zeros = jnp.zeros((NP, DH), f32)

    w1a = W_gc_1[:DH, :]
    w1b = W_gc_1[DH:, :]
    b1 = jnp.broadcast_to(b_gc_1, (8, D))
    w2p = jnp.concatenate([W_gc, jnp.zeros((D, 128 - GR), f32)], axis=1)
    b2p = jnp.broadcast_to(
        jnp.concatenate([b_gc, jnp.full((1, 128 - GR), -1e30, f32)], axis=1),
        (8, 128))

    s0, s1 = _spmm(rows, cols, valsb, e0, e1, zeros)
    G = _route(e0, e1, s0, s1, w1a, w1b, b1, w2p, b2p)

    x8_0, x8_1 = _mask(G, e0, e1)
    ys = [_spmm(rows, cols, valsb, x8_0[g], x8_1[g], zeros) for g in range(GR)]
    y0 = jnp.stack([y[0] for y in ys])
    y1 = jnp.stack([y[1] for y in ys])

    sum1_0, sum1_1, x2_0, x2_1 = _elem1(e0, e1, y0, y1, G)
    c10, c11 = _spmm(rows, cols, valsb, sum1_0, sum1_1, zeros)

    s2s = [_spmm(rows, cols, valsb, x2_0[g], x2_1[g], zeros) for g in range(GR)]
    s2_0 = jnp.stack([s[0] for s in s2s])
    s2_1 = jnp.stack([s[1] for s in s2s])
    sum2_0, sum2_1 = _elem2(s2_0, s2_1, G)
    c20, c21 = _spmm(rows, cols, valsb, sum2_0, sum2_1, zeros)

    f0, f1 = _final(e0, e1, s0, s1, c10, c11, c20, c21)

    idx = jnp.concatenate([users.astype(jnp.int32),
                           NU + pos_items.astype(jnp.int32),
                           NU + neg_items.astype(jnp.int32)])
    go = _gather_kernel()(f0, f1, idx)
    o = jnp.concatenate([go[0], go[1]], axis=1)
    return (o[:B], o[B:2 * B], o[2 * B:])


# speedup vs baseline: 9.3413x; 1.0219x over previous
import functools

import jax
import jax.numpy as jnp
from jax import lax
from jax.experimental import pallas as pl
from jax.experimental.pallas import tpu as pltpu
from jax.experimental.pallas import tpu_sc as plsc

NU = 6000
NI = 4000
NN = 10000
NP = 10240
D = 256
DH = 128
E = 160000
GR = 8
B = 1024
RB = 640
NRB = NP // RB


_TILES = 16
_EPT = E // _TILES
_CHUNK = 80
_NCHUNK = _EPT // _CHUNK
_RPT = NP // _TILES

@functools.lru_cache(maxsize=None)
def _sc_mesh():
    return plsc.VectorSubcoreMesh(core_axis_name="c", subcore_axis_name="s")


def _spmm_body(rows_h, cols_h, vals_h, x0_h, x1_h, zeros_h, out0_h, out1_h,
               idxr0, idxr1, idxc0, idxc1, vals0, vals1, gath0, gath1,
               acc_sh, semg0, semg1, sema0, sema1):
    c = lax.axis_index("c")
    s = lax.axis_index("s")
    rbase = s * _RPT
    ebase = s * _EPT
    idxr = (idxr0, idxr1)
    idxc = (idxc0, idxc1)
    vals = (vals0, vals1)
    gath = (gath0, gath1)
    semg = (semg0, semg1)
    sema = (sema0, sema1)

    def prefetch(b, k):
        off = ebase + k * _CHUNK
        pltpu.sync_copy(cols_h.at[pl.ds(off, _CHUNK)], idxc[b])

        @pl.when(c == 0)
        def _():
            pltpu.async_copy(x0_h.at[idxc[b]], gath[b], semg[b])

        @pl.when(c == 1)
        def _():
            pltpu.async_copy(x1_h.at[idxc[b]], gath[b], semg[b])

        pltpu.async_copy(rows_h.at[pl.ds(off, _CHUNK)], idxr[b], sema[b])
        pltpu.async_copy(vals_h.at[pl.ds(off * 16, _CHUNK * 16)], vals[b], sema[b])

    def wait_bufs(b):
        pltpu.make_async_copy(x0_h.at[idxc[b]], gath[b], semg[b]).wait()
        pltpu.make_async_copy(rows_h.at[pl.ds(0, _CHUNK)], idxr[b], sema[b]).wait()
        pltpu.make_async_copy(vals_h.at[pl.ds(0, _CHUNK * 16)], vals[b], sema[b]).wait()

    def process(b):
        def row_blk(rb, carry2):
            r0 = rb * 16
            for i in range(16):
                v = vals[b][pl.ds((r0 + i) * 16, 16)]
                for d in range(DH // 16):
                    sl = pl.ds(d * 16, 16)
                    gath[b][r0 + i, sl] = gath[b][r0 + i, sl] * v
            return carry2

        lax.fori_loop(0, _CHUNK // 16, row_blk, 0)
        pltpu.sync_copy(gath[b], acc_sh.at[idxr[b]], add=True)

    prefetch(0, 0)
    pltpu.sync_copy(zeros_h.at[pl.ds(rbase, _RPT)], acc_sh.at[pl.ds(rbase, _RPT)])
    plsc.subcore_barrier()

    def pair_body(p, carry):
        wait_bufs(0)
        prefetch(1, 2 * p + 1)
        process(0)
        wait_bufs(1)
        prefetch(0, 2 * p + 2)
        process(1)
        return carry

    lax.fori_loop(0, (_NCHUNK - 1) // 2, pair_body, 0)
    wait_bufs(0)
    process(0)
    plsc.subcore_barrier()

    @pl.when(c == 0)
    def _():
        pltpu.sync_copy(acc_sh.at[pl.ds(rbase, _RPT)], out0_h.at[pl.ds(rbase, _RPT)])

    @pl.when(c == 1)
    def _():
        pltpu.sync_copy(acc_sh.at[pl.ds(rbase, _RPT)], out1_h.at[pl.ds(rbase, _RPT)])


@functools.lru_cache(maxsize=None)
def _spmm_kernel():
    return pl.kernel(
        _spmm_body,
        out_type=[
            jax.ShapeDtypeStruct((NP, DH), jnp.float32),
            jax.ShapeDtypeStruct((NP, DH), jnp.float32),
        ],
        mesh=_sc_mesh(),
        scratch_types=[
            pltpu.VMEM((_CHUNK,), jnp.int32),
            pltpu.VMEM((_CHUNK,), jnp.int32),
            pltpu.VMEM((_CHUNK,), jnp.int32),
            pltpu.VMEM((_CHUNK,), jnp.int32),
            pltpu.VMEM((_CHUNK * 16,), jnp.float32),
            pltpu.VMEM((_CHUNK * 16,), jnp.float32),
            pltpu.VMEM((_CHUNK, DH), jnp.float32),
            pltpu.VMEM((_CHUNK, DH), jnp.float32),
            pltpu.VMEM_SHARED((NP, DH), jnp.float32),
            pltpu.SemaphoreType.DMA,
            pltpu.SemaphoreType.DMA,
            pltpu.SemaphoreType.DMA,
            pltpu.SemaphoreType.DMA,
        ],
    )


def _spmm(rows, cols, valsb, x0, x1, zeros):
    return _spmm_kernel()(rows, cols, valsb, x0, x1, zeros)


def _route_body(e0, e1, s0, s1, w1a, w1b, b1, w2p, b2p, g_ref):
    xa = e0[...] + s0[...]
    xb = e1[...] + s1[...]
    h = (jnp.dot(xa, w1a[...], preferred_element_type=jnp.float32)
         + jnp.dot(xb, w1b[...], preferred_element_type=jnp.float32)
         + b1[0:1, :])
    h = jnp.where(h >= 0, h, 0.01 * h)
    gs = jnp.dot(h, w2p[...], preferred_element_type=jnp.float32) + b2p[0:1, :]
    m = jnp.max(gs, axis=1, keepdims=True)
    g = (gs == m).astype(jnp.float32)
    row = pl.program_id(0) * RB + lax.broadcasted_iota(jnp.int32, (RB, 128), 0)
    g_ref[...] = jnp.where(row < NU, g, 1.0)


def _route(e0, e1, s0, s1, w1a, w1b, b1, w2p, b2p):
    blk = pl.BlockSpec((RB, DH), lambda i: (i, 0))
    full = lambda a: pl.BlockSpec(a.shape, lambda i: tuple(0 for _ in a.shape))
    return pl.pallas_call(
        _route_body,
        grid=(NRB,),
        in_specs=[blk, blk, blk, blk, full(w1a), full(w1b), full(b1),
                  full(w2p), full(b2p)],
        out_specs=pl.BlockSpec((RB, 128), lambda i: (i, 0)),
        out_shape=jax.ShapeDtypeStruct((NP, 128), jnp.float32),
    )(e0, e1, s0, s1, w1a, w1b, b1, w2p, b2p)


def _mask_body(g_ref, e0, e1, x0_ref, x1_ref):
    g = pl.program_id(0)
    onehot = (lax.broadcasted_iota(jnp.int32, (RB, 128), 1) == g).astype(jnp.float32)
    col = jnp.sum(g_ref[...] * onehot, axis=1, keepdims=True)
    x0_ref[0] = col * e0[...]
    x1_ref[0] = col * e1[...]


def _mask(G, e0, e1):
    blk = pl.BlockSpec((RB, DH), lambda g, r: (r, 0))
    gblk = pl.BlockSpec((RB, 128), lambda g, r: (r, 0))
    oblk = pl.BlockSpec((1, RB, DH), lambda g, r: (g, r, 0))
    return pl.pallas_call(
        _mask_body,
        grid=(GR, NRB),
        in_specs=[gblk, blk, blk],
        out_specs=[oblk, oblk],
        out_shape=[jax.ShapeDtypeStruct((GR, NP, DH), jnp.float32),
                   jax.ShapeDtypeStruct((GR, NP, DH), jnp.float32)],
    )(G, e0, e1)


def _elem1_body(e0, e1, y0, y1, g_ref, s1_0, s1_1, x2_0, x2_1):
    g = pl.program_id(1)
    onehot = (lax.broadcasted_iota(jnp.int32, (RB, 128), 1) == g).astype(jnp.float32)
    col = jnp.sum(g_ref[...] * onehot, axis=1, keepdims=True)
    ea = e0[...]
    eb = e1[...]
    mya = col * y0[0]
    myb = col * y1[0]
    fa = ea + mya
    fb = eb + myb
    dot = jnp.sum(fa * ea, axis=1, keepdims=True) + jnp.sum(fb * eb, axis=1, keepdims=True)
    na = jnp.sqrt(jnp.sum(fa * fa, axis=1, keepdims=True) + jnp.sum(fb * fb, axis=1, keepdims=True))
    nb = jnp.sqrt(jnp.sum(ea * ea, axis=1, keepdims=True) + jnp.sum(eb * eb, axis=1, keepdims=True))
    w = dot / (jnp.maximum(na, 1e-8) * jnp.maximum(nb, 1e-8))
    x2_0[0] = w * (col * ea + mya)
    x2_1[0] = w * (col * eb + myb)

    @pl.when(g == 0)
    def _():
        s1_0[...] = mya
        s1_1[...] = myb

    @pl.when(g > 0)
    def _():
        s1_0[...] += mya
        s1_1[...] += myb


def _elem1(e0, e1, y0, y1, G):
    blk = pl.BlockSpec((RB, DH), lambda r, g: (r, 0))
    gblk = pl.BlockSpec((RB, 128), lambda r, g: (r, 0))
    ybk = pl.BlockSpec((1, RB, DH), lambda r, g: (g, r, 0))
    return pl.pallas_call(
        _elem1_body,
        grid=(NRB, GR),
        in_specs=[blk, blk, ybk, ybk, gblk],
        out_specs=[blk, blk, ybk, ybk],
        out_shape=[jax.ShapeDtypeStruct((NP, DH), jnp.float32),
                   jax.ShapeDtypeStruct((NP, DH), jnp.float32),
                   jax.ShapeDtypeStruct((GR, NP, DH), jnp.float32),
                   jax.ShapeDtypeStruct((GR, NP, DH), jnp.float32)],
    )(e0, e1, y0, y1, G)


def _elem2_body(s0, s1, g_ref, o0, o1):
    g = pl.program_id(1)
    onehot = (lax.broadcasted_iota(jnp.int32, (RB, 128), 1) == g).astype(jnp.float32)
    col = jnp.sum(g_ref[...] * onehot, axis=1, keepdims=True)
    va = col * s0[0]
    vb = col * s1[0]

    @pl.when(g == 0)
    def _():
        o0[...] = va
        o1[...] = vb

    @pl.when(g > 0)
    def _():
        o0[...] += va
        o1[...] += vb


def _elem2(s2_0, s2_1, G):
    blk = pl.BlockSpec((RB, DH), lambda r, g: (r, 0))
    gblk = pl.BlockSpec((RB, 128), lambda r, g: (r, 0))
    sbk = pl.BlockSpec((1, RB, DH), lambda r, g: (g, r, 0))
    return pl.pallas_call(
        _elem2_body,
        grid=(NRB, GR),
        in_specs=[sbk, sbk, gblk],
        out_specs=[blk, blk],
        out_shape=[jax.ShapeDtypeStruct((NP, DH), jnp.float32),
                   jax.ShapeDtypeStruct((NP, DH), jnp.float32)],
    )(s2_0, s2_1, G)


def _final_body(e0, e1, s0, s1, c10, c11, c20, c21, f0, f1):
    f0[...] = e0[...] + s0[...] + c10[...] + c20[...]
    f1[...] = e1[...] + s1[...] + c11[...] + c21[...]


def _final(e0, e1, s0, s1, c10, c11, c20, c21):
    blk = pl.BlockSpec((RB, DH), lambda r: (r, 0))
    return pl.pallas_call(
        _final_body,
        grid=(NRB,),
        in_specs=[blk] * 8,
        out_specs=[blk, blk],
        out_shape=[jax.ShapeDtypeStruct((NP, DH), jnp.float32),
                   jax.ShapeDtypeStruct((NP, DH), jnp.float32)],
    )(e0, e1, s0, s1, c10, c11, c20, c21)


_GB = 3 * B
_GPW = _GB // 32


def _gather_body(f0_h, f1_h, idx_h, out_h, idx_v, r0_v, r1_v, sem):
    c = lax.axis_index("c")
    s = lax.axis_index("s")
    base = (s * 2 + c) * _GPW
    pltpu.sync_copy(idx_h.at[pl.ds(base, _GPW)], idx_v)
    pltpu.async_copy(f0_h.at[idx_v], r0_v, sem).wait()
    pltpu.async_copy(f1_h.at[idx_v], r1_v, sem).wait()
    pltpu.sync_copy(r0_v, out_h.at[0, pl.ds(base, _GPW)])
    pltpu.sync_copy(r1_v, out_h.at[1, pl.ds(base, _GPW)])


@functools.lru_cache(maxsize=None)
def _gather_kernel():
    return pl.kernel(
        _gather_body,
        out_type=jax.ShapeDtypeStruct((2, _GB, DH), jnp.float32),
        mesh=_sc_mesh(),
        scratch_types=[
            pltpu.VMEM((_GPW,), jnp.int32),
            pltpu.VMEM((_GPW, DH), jnp.float32),
            pltpu.VMEM((_GPW, DH), jnp.float32),
            pltpu.SemaphoreType.DMA,
        ],
    )


def kernel(users, pos_items, neg_items, user_emb, item_emb,
           W_gc_1, b_gc_1, W_gc, b_gc, adj_rows, adj_cols, adj_vals):
    f32 = jnp.float32
    rows = adj_rows.astype(jnp.int32)
    cols = adj_cols.astype(jnp.int32)
    vals = adj_vals.astype(f32)
    valsb = jnp.reshape(jnp.broadcast_to(vals[:, None], (E, 16)), (E * 16,))

    pad = jnp.zeros((NP - NN, DH), f32)
    e0 = jnp.concatenate([user_emb[:, :DH], item_emb[:, :DH], pad], axis=0)
    e1 = jnp.concatenate([user_emb[:, DH:], item_emb[:, DH:], pad], axis=0)
    zeros = jnp.zeros((NP, DH), f32)

    w1a = W_gc_1[:DH, :]
    w1b = W_gc_1[DH:, :]
    b1 = jnp.broadcast_to(b_gc_1, (8, D))
    w2p = jnp.concatenate([W_gc, jnp.zeros((D, 128 - GR), f32)], axis=1)
    b2p = jnp.broadcast_to(
        jnp.concatenate([b_gc, jnp.full((1, 128 - GR), -1e30, f32)], axis=1),
        (8, 128))

    s0, s1 = _spmm(rows, cols, valsb, e0, e1, zeros)
    G = _route(e0, e1, s0, s1, w1a, w1b, b1, w2p, b2p)

    x8_0, x8_1 = _mask(G, e0, e1)
    ys = [_spmm(rows, cols, valsb, x8_0[g], x8_1[g], zeros) for g in range(GR)]
    y0 = jnp.stack([y[0] for y in ys])
    y1 = jnp.stack([y[1] for y in ys])

    sum1_0, sum1_1, x2_0, x2_1 = _elem1(e0, e1, y0, y1, G)
    c10, c11 = _spmm(rows, cols, valsb, sum1_0, sum1_1, zeros)

    s2s = [_spmm(rows, cols, valsb, x2_0[g], x2_1[g], zeros) for g in range(GR)]
    s2_0 = jnp.stack([s[0] for s in s2s])
    s2_1 = jnp.stack([s[1] for s in s2s])
    sum2_0, sum2_1 = _elem2(s2_0, s2_1, G)
    c20, c21 = _spmm(rows, cols, valsb, sum2_0, sum2_1, zeros)

    f0, f1 = _final(e0, e1, s0, s1, c10, c11, c20, c21)

    idx = jnp.concatenate([users.astype(jnp.int32),
                           NU + pos_items.astype(jnp.int32),
                           NU + neg_items.astype(jnp.int32)])
    go = _gather_kernel()(f0, f1, idx)
    o = jnp.concatenate([go[0], go[1]], axis=1)
    return (o[:B], o[B:2 * B], o[2 * B:])
